# baseline (speedup 1.0000x reference)
import functools
import jax
import jax.numpy as jnp
from jax import lax
from jax.experimental import pallas as pl
from jax.experimental.pallas import tpu as pltpu
from jax.experimental.pallas import tpu_sc as plsc

NC = 2
NS = 16
L = 16
NW = NC * NS

CW = 112


@functools.partial(jax.jit, static_argnums=(3, 4, 5, 6, 7))
def _sc_gather(table, adjc, voff, n_chunks, n_var, C, S, nh):
    rpw = n_chunks * CW
    mesh = plsc.VectorSubcoreMesh(
        core_axis_name="c", subcore_axis_name="s", num_cores=NC, num_subcores=NS
    )

    total = n_var * n_chunks
    nbuf = min(6, total)

    @functools.partial(
        pl.kernel,
        out_type=jax.ShapeDtypeStruct((n_var, S, C), jnp.float32),
        mesh=mesh,
        scratch_types=[
            pltpu.VMEM((n_chunks, CW), jnp.int32),
            pltpu.VMEM((n_chunks, CW), jnp.int32),
            pltpu.VMEM((total, CW), jnp.int32),
            pltpu.VMEM((n_var, L), jnp.int32),
            pltpu.VMEM((nbuf, CW, C), jnp.float32),
            pltpu.SemaphoreType.DMA,
            [pltpu.SemaphoreType.DMA] * nbuf,
            [pltpu.SemaphoreType.DMA] * nbuf,
        ],
    )
    def k(table_h, adjc_h, voff_h, out_h, idx_pos, idx_raw, idx_adj, voff_v,
          rows, csem, gsem, wsem):
        wid = lax.axis_index("s") * NC + lax.axis_index("c")
        start = lax.min(wid * rpw, S - rpw)
        pltpu.sync_copy(voff_h, voff_v)
        iota = lax.iota(jnp.int32, L)
        svec = jnp.broadcast_to(start * nh, (L,))
        for j in range(n_chunks):
            for i in range(CW // L):
                idx_pos[j, pl.ds(i * L, L)] = iota * nh + ((j * CW + i * L) * nh) + svec
        ch = [
            pltpu.async_copy(adjc_h.at[idx_pos.at[j]], idx_raw.at[j], csem)
            for j in range(n_chunks)
        ]
        for h in ch:
            h.wait()
        for v in range(n_var):
            offv = voff_v[v, :]
            for j in range(n_chunks):
                for i in range(CW // L):
                    sl = pl.ds(i * L, L)
                    idx_adj[v * n_chunks + j, sl] = idx_raw[j, sl] + offv
        g_h = [None] * nbuf
        w_h = [None] * nbuf
        for t in range(nbuf):
            g_h[t] = pltpu.async_copy(table_h.at[idx_adj.at[t]], rows.at[t], gsem[t])
        for t in range(total):
            b = t % nbuf
            v, j = divmod(t, n_chunks)
            g_h[b].wait()
            w_h[b] = pltpu.async_copy(
                rows.at[b], out_h.at[v, pl.ds(start + j * CW, CW), :], wsem[b]
            )
            f = t + nbuf
            w_h[b].wait()
            if f < total:
                g_h[b] = pltpu.async_copy(
                    table_h.at[idx_adj.at[f]], rows.at[b], gsem[b]
                )

    return k(table, adjc, voff)


def kernel(mg_emb, var_indices, adjc):
    n_vars, T, S, C = mg_emb.shape
    B, V = var_indices.shape
    n_var = B * V
    nh = adjc.shape[1]

    chunk_rows = NW * CW
    n_chunks = -(-S // chunk_rows)

    table = mg_emb.reshape(n_vars * T * S, C)
    vi = (var_indices.reshape(-1) * (T * S)).astype(jnp.int32)
    voff = jnp.broadcast_to(vi[:, None], (n_var, L))

    adjc_flat = adjc.reshape(-1).astype(jnp.int32)
    out = _sc_gather(table, adjc_flat, voff, n_chunks, n_var, C, S, nh)
    return out.reshape(B, V, T, S, C)


# speedup vs baseline: 1.5246x; 1.5246x over previous
import functools
import jax
import jax.numpy as jnp
import numpy as np
from jax import lax
from jax.experimental import pallas as pl
from jax.experimental.pallas import tpu as pltpu
from jax.experimental.pallas import tpu_sc as plsc

NC = 2
NS = 16
L = 16
NW = NC * NS

CW = 112


@functools.partial(jax.jit, static_argnums=(3, 4, 5, 6))
def _sc_gather(table, col, voff, n_chunks, n_var, C, S):
    rpw = n_chunks * CW
    mesh = plsc.VectorSubcoreMesh(
        core_axis_name="c", subcore_axis_name="s", num_cores=NC, num_subcores=NS
    )

    total = n_var * n_chunks
    nbuf = min(6, total)
    ahead = max(1, nbuf // 2)

    @functools.partial(
        pl.kernel,
        out_type=jax.ShapeDtypeStruct((n_var, S, C), jnp.float32),
        mesh=mesh,
        scratch_types=[
            pltpu.VMEM((rpw,), jnp.int32),
            pltpu.VMEM((total, CW), jnp.int32),
            pltpu.VMEM((n_var, L), jnp.int32),
            pltpu.VMEM((nbuf, CW, C), jnp.float32),
            [pltpu.SemaphoreType.DMA] * nbuf,
            [pltpu.SemaphoreType.DMA] * nbuf,
        ],
    )
    def k(table_h, adjc_h, voff_h, out_h, idx_raw, idx_adj, voff_v, rows, gsem, wsem):
        wid = lax.axis_index("s") * NC + lax.axis_index("c")
        start = lax.min(wid * rpw, S - rpw)
        pltpu.sync_copy(adjc_h.at[pl.ds(start, rpw)], idx_raw)
        pltpu.sync_copy(voff_h, voff_v)
        g = [None] * nbuf
        w = [None] * nbuf
        for t in range(total):
            v, j = divmod(t, n_chunks)
            off = voff_v[v, :]
            for i in range(CW // L):
                sl = pl.ds(i * L, L)
                idx_adj[t, sl] = idx_raw[pl.ds(j * CW + i * L, L)] + off
            if t < ahead:
                g[t % nbuf] = pltpu.async_copy(
                    table_h.at[idx_adj.at[t]], rows.at[t % nbuf], gsem[t % nbuf]
                )
        for t in range(total):
            b = t % nbuf
            v, j = divmod(t, n_chunks)
            g[b].wait()
            w[b] = pltpu.async_copy(
                rows.at[b], out_h.at[v, pl.ds(start + j * CW, CW), :], wsem[b]
            )
            f = t + ahead
            if f < total:
                fb = f % nbuf
                if w[fb] is not None:
                    w[fb].wait()
                    w[fb] = None
                g[fb] = pltpu.async_copy(
                    table_h.at[idx_adj.at[f]], rows.at[fb], gsem[fb]
                )
        for b in range(nbuf):
            if w[b] is not None:
                w[b].wait()

    return k(table, col, voff)


def kernel(mg_emb, var_indices, adjc):
    n_vars, T, S, C = mg_emb.shape
    B, V = var_indices.shape
    n_var = B * V

    chunk_rows = NW * CW
    n_chunks = -(-S // chunk_rows)
    rpw = n_chunks * CW

    table = mg_emb.reshape(n_vars * T * S, C)
    vi = var_indices.reshape(-1).astype(jnp.int32)
    voff = jnp.broadcast_to((vi * (T * S))[:, None], (n_var, L))

    col = adjc[:, 0].astype(jnp.int32)
    out = _sc_gather(table, col, voff, n_chunks, n_var, C, S)
    return out.reshape(B, V, T, S, C)


# speedup vs baseline: 1.5282x; 1.0024x over previous
import functools
import jax
import jax.numpy as jnp
import numpy as np
from jax import lax
from jax.experimental import pallas as pl
from jax.experimental.pallas import tpu as pltpu
from jax.experimental.pallas import tpu_sc as plsc

NC = 2
NS = 16
L = 16
NW = NC * NS

CW = 112


@functools.partial(jax.jit, static_argnums=(3, 4, 5, 6))
def _sc_gather(table, col, voff, n_chunks, n_var, C, S):
    rpw = n_chunks * CW
    mesh = plsc.VectorSubcoreMesh(
        core_axis_name="c", subcore_axis_name="s", num_cores=NC, num_subcores=NS
    )

    total = n_var * n_chunks
    nbuf = min(6, total)
    ahead = max(1, nbuf // 2)

    @functools.partial(
        pl.kernel,
        out_type=jax.ShapeDtypeStruct((n_var, S, C), jnp.float32),
        mesh=mesh,
        scratch_types=[
            pltpu.VMEM((rpw,), jnp.int32),
            pltpu.VMEM((total, CW), jnp.int32),
            pltpu.VMEM((n_var, L), jnp.int32),
            pltpu.VMEM((nbuf, CW, C), jnp.float32),
            [pltpu.SemaphoreType.DMA] * nbuf,
            [pltpu.SemaphoreType.DMA] * nbuf,
        ],
    )
    def k(table_h, adjc_h, voff_h, out_h, idx_raw, idx_adj, voff_v, rows, gsem, wsem):
        wid = lax.axis_index("s") * NC + lax.axis_index("c")
        start = lax.min(wid * rpw, S - rpw)
        pltpu.sync_copy(adjc_h.at[pl.ds(start, rpw)], idx_raw)
        pltpu.sync_copy(voff_h, voff_v)
        g = [None] * nbuf
        w = [None] * nbuf
        spc = CW // L
        for t in range(ahead):
            v, j = divmod(t, n_chunks)
            off = voff_v[v, :]
            for i in range(spc):
                sl = pl.ds(i * L, L)
                idx_adj[t, sl] = idx_raw[pl.ds(j * CW + i * L, L)] + off
            g[t % nbuf] = pltpu.async_copy(
                table_h.at[idx_adj.at[t]], rows.at[t % nbuf], gsem[t % nbuf]
            )
        for v in range(n_var):
            off = voff_v[v, :]

            def idx_body(k, _, off=off, v=v):
                lo = max(ahead - v * n_chunks, 0) * spc
                src = pl.ds(lo * L + k * L, L)
                dst_row = v * n_chunks + (lo + k) // spc
                dst_sl = pl.ds(((lo + k) % spc) * L, L)
                idx_adj[dst_row, dst_sl] = idx_raw[src] + off
                return _

            n_sl = n_chunks * spc - max(ahead - v * n_chunks, 0) * spc
            lax.fori_loop(0, n_sl, idx_body, None)
        for t in range(total):
            b = t % nbuf
            v, j = divmod(t, n_chunks)
            g[b].wait()
            w[b] = pltpu.async_copy(
                rows.at[b], out_h.at[v, pl.ds(start + j * CW, CW), :], wsem[b]
            )
            f = t + ahead
            if f < total:
                fb = f % nbuf
                if w[fb] is not None:
                    w[fb].wait()
                    w[fb] = None
                g[fb] = pltpu.async_copy(
                    table_h.at[idx_adj.at[f]], rows.at[fb], gsem[fb]
                )
        for b in range(nbuf):
            if w[b] is not None:
                w[b].wait()

    return k(table, col, voff)


def kernel(mg_emb, var_indices, adjc):
    n_vars, T, S, C = mg_emb.shape
    B, V = var_indices.shape
    n_var = B * V

    chunk_rows = NW * CW
    n_chunks = -(-S // chunk_rows)
    rpw = n_chunks * CW

    table = mg_emb.reshape(n_vars * T * S, C)
    vi = var_indices.reshape(-1).astype(jnp.int32)
    voff = jnp.broadcast_to((vi * (T * S))[:, None], (n_var, L))

    col = adjc[:, 0].astype(jnp.int32)
    out = _sc_gather(table, col, voff, n_chunks, n_var, C, S)
    return out.reshape(B, V, T, S, C)
